# baseline (speedup 1.0000x reference)
import jax
import jax.numpy as jnp
from jax.experimental import pallas as pl


def _to_quat(x):
    z = jnp.zeros(x.shape[:-1] + (1,), x.dtype)
    return jnp.concatenate([z, x], axis=-1)[..., None, :]


def _fps(xyz, npoint):
    B, N, _ = xyz.shape
    def body(i, state):
        centroids, distance, farthest = state
        centroids = centroids.at[:, i].set(farthest)
        centroid = jnp.take_along_axis(xyz, farthest[:, None, None], axis=1)
        dist = jnp.sum((xyz - centroid) ** 2, -1)
        distance = jnp.minimum(distance, dist)
        farthest = jnp.argmax(distance, -1).astype(jnp.int32)
        return (centroids, distance, farthest)
    init = (jnp.zeros((B, npoint), jnp.int32), jnp.full((B, N), 1e10, xyz.dtype), jnp.zeros((B,), jnp.int32))
    centroids, _, _ = jax.lax.fori_loop(0, npoint, body, init)
    return centroids


def _query_ball(radius, nsample, xyz, new_xyz):
    B, S, _ = new_xyz.shape
    N = xyz.shape[1]
    sqr = jnp.sum((new_xyz[:, :, None, :] - xyz[:, None, :, :]) ** 2, -1)
    gidx = jnp.broadcast_to(jnp.arange(N, dtype=jnp.int32), (B, S, N))
    gidx = jnp.where(sqr > radius ** 2, N, gidx)
    gidx = jnp.sort(gidx, axis=-1)[:, :, :nsample]
    first = gidx[:, :, :1]
    gidx = jnp.where(gidx == N, jnp.broadcast_to(first, gidx.shape), gidx)
    return gidx


def _gather(x, idx):
    B, N = x.shape[0], x.shape[1]
    flat = idx.reshape(B, -1)
    out = jnp.take_along_axis(x.reshape(B, N, -1), flat[:, :, None], axis=1)
    return out.reshape(B, idx.shape[1], idx.shape[2], *x.shape[2:])


def _q_mlp(feat, ws, gs, ts):
    for W, g, t in zip(ws, gs, ts):
        feat = jnp.einsum('oc,...ci->...oi', W, feat)
        n2 = jnp.sum(feat ** 2, -1)
        red = tuple(range(n2.ndim - 1))
        mean = jnp.mean(n2, axis=red)
        feat = feat / jnp.sqrt(mean + 1e-5)[:, None] * g[:, None]
        norm = jnp.sqrt(jnp.sum(feat ** 2, -1) + 1e-8)
        scale = jnp.minimum(1.0, norm / t)
        feat = feat * scale[..., None]
    return feat


def _sa(xyz, points, npoint, radius, nsample, ws, gs, ts, group_all):
    B, N, _ = xyz.shape
    if group_all:
        new_xyz = jnp.zeros((B, 1, 3), xyz.dtype)
        chans = [_to_quat(xyz[:, None, :, :])]
        if points is not None:
            chans.append(points[:, None])
    else:
        fps_idx = _fps(jax.lax.stop_gradient(xyz), npoint)
        new_xyz = jnp.take_along_axis(xyz, fps_idx[:, :, None], axis=1)
        gidx = _query_ball(radius, nsample, xyz, new_xyz)
        grouped = _gather(xyz, gidx)
        chans = [_to_quat(grouped - new_xyz[:, :, None, :]), _to_quat(grouped)]
        if points is not None:
            chans.append(_gather(points, gidx))
    feat = jnp.concatenate(chans, axis=3)
    feat = _q_mlp(feat, ws, gs, ts)
    norms = jnp.sqrt(jnp.sum(feat ** 2, -1) + 1e-8)
    idx = jnp.argmax(norms, axis=2)
    pooled = jnp.take_along_axis(feat, idx[:, :, None, :, None], axis=2)[:, :, 0]
    return new_xyz, pooled


def _bn(v, g, b):
    m = jnp.mean(v, 0)
    var = jnp.var(v, 0)
    return (v - m) / jnp.sqrt(var + 1e-5) * g + b


def kernel(xyz, params):
    B = xyz.shape[0]
    x = jnp.transpose(xyz, (0, 2, 1))
    l1_xyz, l1_p = _sa(x, None, 512, 0.2, 32, params['sa1_w'], params['sa1_g'], params['sa1_t'], False)
    l2_xyz, l2_p = _sa(l1_xyz, l1_p, 128, 0.4, 64, params['sa2_w'], params['sa2_g'], params['sa2_t'], False)
    _, l3_p = _sa(l2_xyz, l2_p, None, None, 128, params['sa3_w'], params['sa3_g'], params['sa3_t'], True)
    h = jnp.sqrt(jnp.sum(l3_p ** 2, -1) + 1e-8).reshape(B, 1024)
    h = jax.nn.leaky_relu(_bn(h @ params['fc1_w'].T + params['fc1_b'], params['bn1_g'], params['bn1_b']), 0.2)
    h = jax.nn.leaky_relu(_bn(h @ params['fc2_w'].T + params['fc2_b'], params['bn2_g'], params['bn2_b']), 0.2)
    return h @ params['fc3_w'].T + params['fc3_b']


# speedup vs baseline: 1.4788x; 1.4788x over previous
import functools

import jax
import jax.numpy as jnp
from jax.experimental import pallas as pl


def _fps_body(npoint, x_ref, y_ref, z_ref, cx_ref, cy_ref, cz_ref, idx_ref):
    B, N = x_ref.shape
    S = npoint
    x = x_ref[...]
    y = y_ref[...]
    z = z_ref[...]
    iota_n = jax.lax.broadcasted_iota(jnp.int32, (B, N), 1)
    iota_s = jax.lax.broadcasted_iota(jnp.int32, (B, S), 1)

    def body(i, st):
        dist, far, idxs, cxs, cys, czs = st
        onehot = iota_n == far
        cx = jnp.sum(jnp.where(onehot, x, 0.0), axis=1, keepdims=True)
        cy = jnp.sum(jnp.where(onehot, y, 0.0), axis=1, keepdims=True)
        cz = jnp.sum(jnp.where(onehot, z, 0.0), axis=1, keepdims=True)
        sel = iota_s == i
        idxs = jnp.where(sel, far, idxs)
        cxs = jnp.where(sel, cx, cxs)
        cys = jnp.where(sel, cy, cys)
        czs = jnp.where(sel, cz, czs)
        d = (x - cx) ** 2 + (y - cy) ** 2 + (z - cz) ** 2
        dist = jnp.minimum(dist, d)
        far = jnp.argmax(dist, axis=1, keepdims=True).astype(jnp.int32)
        return (dist, far, idxs, cxs, cys, czs)

    zs = jnp.minimum(iota_s, 0)
    zsf = zs.astype(jnp.float32)
    init = (
        jnp.maximum(iota_n.astype(jnp.float32), 1e10),
        jnp.minimum(jax.lax.broadcasted_iota(jnp.int32, (B, 1), 0), 0),
        zs,
        zsf,
        zsf,
        zsf,
    )
    _, _, idxs, cxs, cys, czs = jax.lax.fori_loop(0, npoint, body, init)
    cx_ref[...] = cxs
    cy_ref[...] = cys
    cz_ref[...] = czs
    idx_ref[...] = idxs


def _fps_pallas(x, y, z, npoint):
    B, N = x.shape
    out_shape = (
        jax.ShapeDtypeStruct((B, npoint), jnp.float32),
        jax.ShapeDtypeStruct((B, npoint), jnp.float32),
        jax.ShapeDtypeStruct((B, npoint), jnp.float32),
        jax.ShapeDtypeStruct((B, npoint), jnp.int32),
    )
    return pl.pallas_call(
        functools.partial(_fps_body, npoint),
        out_shape=out_shape,
    )(x, y, z)


def _to_quat(x):
    z = jnp.zeros(x.shape[:-1] + (1,), x.dtype)
    return jnp.concatenate([z, x], axis=-1)[..., None, :]


def _fps(xyz, npoint):
    B, N, _ = xyz.shape
    def body(i, state):
        centroids, distance, farthest = state
        centroids = centroids.at[:, i].set(farthest)
        centroid = jnp.take_along_axis(xyz, farthest[:, None, None], axis=1)
        dist = jnp.sum((xyz - centroid) ** 2, -1)
        distance = jnp.minimum(distance, dist)
        farthest = jnp.argmax(distance, -1).astype(jnp.int32)
        return (centroids, distance, farthest)
    init = (jnp.zeros((B, npoint), jnp.int32), jnp.full((B, N), 1e10, xyz.dtype), jnp.zeros((B,), jnp.int32))
    centroids, _, _ = jax.lax.fori_loop(0, npoint, body, init)
    return centroids


def _query_ball(radius, nsample, xyz, new_xyz):
    B, S, _ = new_xyz.shape
    N = xyz.shape[1]
    sqr = jnp.sum((new_xyz[:, :, None, :] - xyz[:, None, :, :]) ** 2, -1)
    gidx = jnp.broadcast_to(jnp.arange(N, dtype=jnp.int32), (B, S, N))
    gidx = jnp.where(sqr > radius ** 2, N, gidx)
    gidx = jnp.sort(gidx, axis=-1)[:, :, :nsample]
    first = gidx[:, :, :1]
    gidx = jnp.where(gidx == N, jnp.broadcast_to(first, gidx.shape), gidx)
    return gidx


def _gather(x, idx):
    B, N = x.shape[0], x.shape[1]
    flat = idx.reshape(B, -1)
    out = jnp.take_along_axis(x.reshape(B, N, -1), flat[:, :, None], axis=1)
    return out.reshape(B, idx.shape[1], idx.shape[2], *x.shape[2:])


def _q_mlp(feat, ws, gs, ts):
    for W, g, t in zip(ws, gs, ts):
        feat = jnp.einsum('oc,...ci->...oi', W, feat)
        n2 = jnp.sum(feat ** 2, -1)
        red = tuple(range(n2.ndim - 1))
        mean = jnp.mean(n2, axis=red)
        feat = feat / jnp.sqrt(mean + 1e-5)[:, None] * g[:, None]
        norm = jnp.sqrt(jnp.sum(feat ** 2, -1) + 1e-8)
        scale = jnp.minimum(1.0, norm / t)
        feat = feat * scale[..., None]
    return feat


def _sa(xyz, points, npoint, radius, nsample, ws, gs, ts, group_all):
    B, N, _ = xyz.shape
    if group_all:
        new_xyz = jnp.zeros((B, 1, 3), xyz.dtype)
        chans = [_to_quat(xyz[:, None, :, :])]
        if points is not None:
            chans.append(points[:, None])
    else:
        cx, cy, cz, _ = _fps_pallas(xyz[..., 0], xyz[..., 1], xyz[..., 2], npoint)
        new_xyz = jnp.stack([cx, cy, cz], axis=-1)
        gidx = _query_ball(radius, nsample, xyz, new_xyz)
        grouped = _gather(xyz, gidx)
        chans = [_to_quat(grouped - new_xyz[:, :, None, :]), _to_quat(grouped)]
        if points is not None:
            chans.append(_gather(points, gidx))
    feat = jnp.concatenate(chans, axis=3)
    feat = _q_mlp(feat, ws, gs, ts)
    norms = jnp.sqrt(jnp.sum(feat ** 2, -1) + 1e-8)
    idx = jnp.argmax(norms, axis=2)
    pooled = jnp.take_along_axis(feat, idx[:, :, None, :, None], axis=2)[:, :, 0]
    return new_xyz, pooled


def _bn(v, g, b):
    m = jnp.mean(v, 0)
    var = jnp.var(v, 0)
    return (v - m) / jnp.sqrt(var + 1e-5) * g + b


def kernel(xyz, params):
    B = xyz.shape[0]
    x = jnp.transpose(xyz, (0, 2, 1))
    l1_xyz, l1_p = _sa(x, None, 512, 0.2, 32, params['sa1_w'], params['sa1_g'], params['sa1_t'], False)
    l2_xyz, l2_p = _sa(l1_xyz, l1_p, 128, 0.4, 64, params['sa2_w'], params['sa2_g'], params['sa2_t'], False)
    _, l3_p = _sa(l2_xyz, l2_p, None, None, 128, params['sa3_w'], params['sa3_g'], params['sa3_t'], True)
    h = jnp.sqrt(jnp.sum(l3_p ** 2, -1) + 1e-8).reshape(B, 1024)
    h = jax.nn.leaky_relu(_bn(h @ params['fc1_w'].T + params['fc1_b'], params['bn1_g'], params['bn1_b']), 0.2)
    h = jax.nn.leaky_relu(_bn(h @ params['fc2_w'].T + params['fc2_b'], params['bn2_g'], params['bn2_b']), 0.2)
    return h @ params['fc3_w'].T + params['fc3_b']


# speedup vs baseline: 1.5301x; 1.0347x over previous
import functools

import jax
import jax.numpy as jnp
from jax.experimental import pallas as pl


def _fps_body(npoint, x_ref, y_ref, z_ref, cx_ref, cy_ref, cz_ref, idx_ref):
    B, N = x_ref.shape
    S = npoint
    x = x_ref[...]
    y = y_ref[...]
    z = z_ref[...]
    iota_n = jax.lax.broadcasted_iota(jnp.int32, (B, N), 1)
    iota_s = jax.lax.broadcasted_iota(jnp.int32, (B, S), 1)

    def body(i, st):
        dist, far, idxs, cxs, cys, czs = st
        onehot = iota_n == far
        cx = jnp.sum(jnp.where(onehot, x, 0.0), axis=1, keepdims=True)
        cy = jnp.sum(jnp.where(onehot, y, 0.0), axis=1, keepdims=True)
        cz = jnp.sum(jnp.where(onehot, z, 0.0), axis=1, keepdims=True)
        sel = iota_s == i
        idxs = jnp.where(sel, far, idxs)
        cxs = jnp.where(sel, cx, cxs)
        cys = jnp.where(sel, cy, cys)
        czs = jnp.where(sel, cz, czs)
        d = (x - cx) ** 2 + (y - cy) ** 2 + (z - cz) ** 2
        dist = jnp.minimum(dist, d)
        far = jnp.argmax(dist, axis=1, keepdims=True).astype(jnp.int32)
        return (dist, far, idxs, cxs, cys, czs)

    zs = jnp.minimum(iota_s, 0)
    zsf = zs.astype(jnp.float32)
    init = (
        jnp.maximum(iota_n.astype(jnp.float32), 1e10),
        jnp.minimum(jax.lax.broadcasted_iota(jnp.int32, (B, 1), 0), 0),
        zs,
        zsf,
        zsf,
        zsf,
    )
    _, _, idxs, cxs, cys, czs = jax.lax.fori_loop(0, npoint, body, init)
    cx_ref[...] = cxs
    cy_ref[...] = cys
    cz_ref[...] = czs
    idx_ref[...] = idxs


def _fps_pallas(x, y, z, npoint):
    B, N = x.shape
    out_shape = (
        jax.ShapeDtypeStruct((B, npoint), jnp.float32),
        jax.ShapeDtypeStruct((B, npoint), jnp.float32),
        jax.ShapeDtypeStruct((B, npoint), jnp.float32),
        jax.ShapeDtypeStruct((B, npoint), jnp.int32),
    )
    return pl.pallas_call(
        functools.partial(_fps_body, npoint),
        out_shape=out_shape,
    )(x, y, z)


def _qb_body(r2, nsample, N, x_ref, y_ref, z_ref, cx_ref, cy_ref, cz_ref, out_ref):
    S = cx_ref.shape[1]
    x = x_ref[0]
    y = y_ref[0]
    z = z_ref[0]
    cx = cx_ref[0]
    cy = cy_ref[0]
    cz = cz_ref[0]
    d = (cx - x) ** 2 + (cy - y) ** 2 + (cz - z) ** 2
    iota_n = jax.lax.broadcasted_iota(jnp.int32, (S, N), 1)
    midx = jnp.where(d > r2, N, iota_n)
    iota_k = jax.lax.broadcasted_iota(jnp.int32, (S, nsample), 1)

    def step(k, st):
        mi, out = st
        m = jnp.min(mi, axis=1, keepdims=True)
        out = jnp.where(iota_k == k, m, out)
        mi = jnp.where(mi == m, N, mi)
        return mi, out

    _, out = jax.lax.fori_loop(
        0, nsample, step, (midx, jnp.maximum(iota_k, N)))
    out_ref[0] = jnp.where(out == N, out[:, 0:1], out)


def _qb_pallas(x, y, z, cx, cy, cz, radius, nsample):
    B, N = x.shape
    S = cx.shape[1]
    return pl.pallas_call(
        functools.partial(_qb_body, radius ** 2, nsample, N),
        grid=(B,),
        in_specs=[
            pl.BlockSpec((1, 1, N), lambda b: (b, 0, 0)),
            pl.BlockSpec((1, 1, N), lambda b: (b, 0, 0)),
            pl.BlockSpec((1, 1, N), lambda b: (b, 0, 0)),
            pl.BlockSpec((1, S, 1), lambda b: (b, 0, 0)),
            pl.BlockSpec((1, S, 1), lambda b: (b, 0, 0)),
            pl.BlockSpec((1, S, 1), lambda b: (b, 0, 0)),
        ],
        out_specs=pl.BlockSpec((1, S, nsample), lambda b: (b, 0, 0)),
        out_shape=jax.ShapeDtypeStruct((B, S, nsample), jnp.int32),
    )(x[:, None, :], y[:, None, :], z[:, None, :],
      cx[:, :, None], cy[:, :, None], cz[:, :, None])


def _to_quat(x):
    z = jnp.zeros(x.shape[:-1] + (1,), x.dtype)
    return jnp.concatenate([z, x], axis=-1)[..., None, :]


def _fps(xyz, npoint):
    B, N, _ = xyz.shape
    def body(i, state):
        centroids, distance, farthest = state
        centroids = centroids.at[:, i].set(farthest)
        centroid = jnp.take_along_axis(xyz, farthest[:, None, None], axis=1)
        dist = jnp.sum((xyz - centroid) ** 2, -1)
        distance = jnp.minimum(distance, dist)
        farthest = jnp.argmax(distance, -1).astype(jnp.int32)
        return (centroids, distance, farthest)
    init = (jnp.zeros((B, npoint), jnp.int32), jnp.full((B, N), 1e10, xyz.dtype), jnp.zeros((B,), jnp.int32))
    centroids, _, _ = jax.lax.fori_loop(0, npoint, body, init)
    return centroids


def _query_ball(radius, nsample, xyz, new_xyz):
    B, S, _ = new_xyz.shape
    N = xyz.shape[1]
    sqr = jnp.sum((new_xyz[:, :, None, :] - xyz[:, None, :, :]) ** 2, -1)
    gidx = jnp.broadcast_to(jnp.arange(N, dtype=jnp.int32), (B, S, N))
    gidx = jnp.where(sqr > radius ** 2, N, gidx)
    gidx = jnp.sort(gidx, axis=-1)[:, :, :nsample]
    first = gidx[:, :, :1]
    gidx = jnp.where(gidx == N, jnp.broadcast_to(first, gidx.shape), gidx)
    return gidx


def _gather(x, idx):
    B, N = x.shape[0], x.shape[1]
    flat = idx.reshape(B, -1)
    out = jnp.take_along_axis(x.reshape(B, N, -1), flat[:, :, None], axis=1)
    return out.reshape(B, idx.shape[1], idx.shape[2], *x.shape[2:])


def _q_mlp(feat, ws, gs, ts):
    for W, g, t in zip(ws, gs, ts):
        feat = jnp.einsum('oc,...ci->...oi', W, feat)
        n2 = jnp.sum(feat ** 2, -1)
        red = tuple(range(n2.ndim - 1))
        mean = jnp.mean(n2, axis=red)
        feat = feat / jnp.sqrt(mean + 1e-5)[:, None] * g[:, None]
        norm = jnp.sqrt(jnp.sum(feat ** 2, -1) + 1e-8)
        scale = jnp.minimum(1.0, norm / t)
        feat = feat * scale[..., None]
    return feat


def _sa(xyz, points, npoint, radius, nsample, ws, gs, ts, group_all):
    B, N, _ = xyz.shape
    if group_all:
        new_xyz = jnp.zeros((B, 1, 3), xyz.dtype)
        chans = [_to_quat(xyz[:, None, :, :])]
        if points is not None:
            chans.append(points[:, None])
    else:
        xp, yp, zp = xyz[..., 0], xyz[..., 1], xyz[..., 2]
        cx, cy, cz, _ = _fps_pallas(xp, yp, zp, npoint)
        new_xyz = jnp.stack([cx, cy, cz], axis=-1)
        gidx = _qb_pallas(xp, yp, zp, cx, cy, cz, radius, nsample)
        grouped = _gather(xyz, gidx)
        chans = [_to_quat(grouped - new_xyz[:, :, None, :]), _to_quat(grouped)]
        if points is not None:
            chans.append(_gather(points, gidx))
    feat = jnp.concatenate(chans, axis=3)
    feat = _q_mlp(feat, ws, gs, ts)
    norms = jnp.sqrt(jnp.sum(feat ** 2, -1) + 1e-8)
    idx = jnp.argmax(norms, axis=2)
    pooled = jnp.take_along_axis(feat, idx[:, :, None, :, None], axis=2)[:, :, 0]
    return new_xyz, pooled


def _bn(v, g, b):
    m = jnp.mean(v, 0)
    var = jnp.var(v, 0)
    return (v - m) / jnp.sqrt(var + 1e-5) * g + b


def kernel(xyz, params):
    B = xyz.shape[0]
    x = jnp.transpose(xyz, (0, 2, 1))
    l1_xyz, l1_p = _sa(x, None, 512, 0.2, 32, params['sa1_w'], params['sa1_g'], params['sa1_t'], False)
    l2_xyz, l2_p = _sa(l1_xyz, l1_p, 128, 0.4, 64, params['sa2_w'], params['sa2_g'], params['sa2_t'], False)
    _, l3_p = _sa(l2_xyz, l2_p, None, None, 128, params['sa3_w'], params['sa3_g'], params['sa3_t'], True)
    h = jnp.sqrt(jnp.sum(l3_p ** 2, -1) + 1e-8).reshape(B, 1024)
    h = jax.nn.leaky_relu(_bn(h @ params['fc1_w'].T + params['fc1_b'], params['bn1_g'], params['bn1_b']), 0.2)
    h = jax.nn.leaky_relu(_bn(h @ params['fc2_w'].T + params['fc2_b'], params['bn2_g'], params['bn2_b']), 0.2)
    return h @ params['fc3_w'].T + params['fc3_b']
